# baseline (device time: 27918 ns/iter reference)
import jax
import jax.numpy as jnp
from jax import lax
from jax.experimental import pallas as pl
from jax.experimental.pallas import tpu as pltpu

N_DEV = 8
B, SQ, SKV, DH = 2, 256, 256, 64
H_PER = 4
DM = 512
HD = H_PER * DH
BLK = 64
ROWS = B * SQ
CH = ROWS // N_DEV

def T(m: int) -> int:
    return (m & 6) | ((m ^ (m >> 1)) & 1)


def kernel(x, Wq, K_ext, V_ext, Wo):
    def body(x_ref, wq_ref, k_ref, v_ref, wo_ref, out_ref,
             pscratch_ref, comm_ref, stg_ref, send_sems, recv_sems):
        my = lax.axis_index("i")
        vr = (my & 6) | ((my ^ (my >> 1)) & 1)

        barrier_sem = pltpu.get_barrier_semaphore()
        for m in range(1, N_DEV):
            pl.semaphore_signal(
                barrier_sem, inc=1,
                device_id=(my ^ m,), device_id_type=pl.DeviceIdType.MESH,
            )
        pl.semaphore_wait(barrier_sem, N_DEV - 1)

        xm = x_ref[...].reshape(ROWS, DM).astype(jnp.bfloat16)
        wq = wq_ref[:, pl.ds(my * HD, HD)].astype(jnp.bfloat16)
        q = jnp.dot(xm, wq, preferred_element_type=jnp.float32)
        qs = (q * 0.125).astype(jnp.bfloat16)

        qb = lax.broadcasted_iota(jnp.int32, (SQ, SKV), 0) // BLK
        kb = lax.broadcasted_iota(jnp.int32, (SQ, SKV), 1) // BLK
        mask = (qb == kb) | (kb == 0) | ((qb + kb) % 3 == 0)

        kv_k = k_ref[...].reshape(B, SQ, HD).astype(jnp.bfloat16)
        kv_v = v_ref[...].reshape(B, SQ, HD).astype(jnp.bfloat16)
        ctx_rows = []
        for b in range(B):
            head_cols = []
            for h in range(H_PER):
                qh = qs[b * SQ:(b + 1) * SQ, h * DH:(h + 1) * DH]
                kh = kv_k[b, :, h * DH:(h + 1) * DH]
                vh = kv_v[b, :, h * DH:(h + 1) * DH]
                s = lax.dot_general(
                    qh, kh, (((1,), (1,)), ((), ())),
                    preferred_element_type=jnp.float32,
                )
                w = jnp.exp(jnp.where(mask, s, -1e9))
                u = jnp.dot(w.astype(jnp.bfloat16), vh,
                            preferred_element_type=jnp.float32)
                head_cols.append(u / jnp.sum(w, axis=-1, keepdims=True))
            ctx_rows.append(jnp.concatenate(head_cols, axis=1))
        ctx = jnp.concatenate(ctx_rows, axis=0)

        wo = wo_ref[pl.ds(my * HD, HD), :].astype(jnp.bfloat16)
        pscratch_ref[...] = jnp.dot(
            ctx.astype(jnp.bfloat16), wo,
            preferred_element_type=jnp.float32)

        for s in range(N_DEV):
            g = (vr ^ s) * CH
            comm_ref[s * CH:(s + 1) * CH, :] = (
                pscratch_ref[pl.ds(g, CH), :].astype(jnp.bfloat16))

        rs = []
        for m in range(1, N_DEV):
            rs.append(pltpu.make_async_remote_copy(
                src_ref=comm_ref.at[pl.ds(T(m) * CH, CH)],
                dst_ref=stg_ref.at[pl.ds((m - 1) * CH, CH)],
                send_sem=send_sems.at[m - 1],
                recv_sem=recv_sems.at[m - 1],
                device_id=(my ^ m,),
                device_id_type=pl.DeviceIdType.MESH,
            ))
        for r in rs:
            r.start()
        total = comm_ref[0:CH, :].astype(jnp.float32)
        for m in range(1, N_DEV):
            rs[m - 1].wait()
            total = total + stg_ref[(m - 1) * CH:m * CH, :].astype(
                jnp.float32)
        comm_ref[0:CH, :] = total.astype(jnp.bfloat16)

        ag = []
        for m in range(1, N_DEV):
            ag.append(pltpu.make_async_remote_copy(
                src_ref=comm_ref.at[pl.ds(0, CH)],
                dst_ref=comm_ref.at[pl.ds(T(m) * CH, CH)],
                send_sem=send_sems.at[N_DEV - 1 + m - 1],
                recv_sem=recv_sems.at[N_DEV - 1 + m - 1],
                device_id=(my ^ m,),
                device_id_type=pl.DeviceIdType.MESH,
            ))
        for r in ag:
            r.start()
        out_ref[pl.ds(vr * CH, CH), :] = (
            comm_ref[0:CH, :].astype(jnp.float32))
        for m in range(1, N_DEV):
            ag[m - 1].wait()
            s = T(m)
            g = (vr ^ s) * CH
            out_ref[pl.ds(g, CH), :] = (
                comm_ref[s * CH:(s + 1) * CH, :].astype(jnp.float32))

    out = pl.pallas_call(
        body,
        out_shape=jax.ShapeDtypeStruct((ROWS, DM), jnp.float32),
        in_specs=[pl.BlockSpec(memory_space=pltpu.VMEM)] * 5,
        out_specs=pl.BlockSpec(memory_space=pltpu.VMEM),
        scratch_shapes=[
            pltpu.VMEM((ROWS, DM), jnp.float32),
            pltpu.VMEM((ROWS, DM), jnp.bfloat16),
            pltpu.VMEM((448, DM), jnp.bfloat16),
            pltpu.SemaphoreType.DMA((14,)),
            pltpu.SemaphoreType.DMA((14,)),
        ],
        compiler_params=pltpu.CompilerParams(collective_id=0),
    )(x, Wq, K_ext, V_ext, Wo)
    return out.reshape(B, SQ, DM)
